# baseline (speedup 1.0000x reference)
import functools

import jax
import jax.numpy as jnp
from jax import lax
from jax.experimental import pallas as pl
from jax.experimental.pallas import tpu as pltpu
from jax.experimental.pallas import tpu_sc as plsc

N_NODES = 10000
N_EDGES = 320000
F = 128
NUM_REL = 8
NUM_BASES = 4

NODE_BLK = 1000

NC = 2
NS = 16
NW = NC * NS
EDGES_PER_W = N_EDGES // NW
CHUNK = 80
N_CHUNKS = EDGES_PER_W // CHUNK
ROWS_MAIN = 624
ROWS_TAIL = N_NODES - NS * ROWS_MAIN
ET_BLK = 2000


def _z_body(coeff_ref, bases_ref, x_ref, z_ref):
    xb = x_ref[...]
    for r in range(NUM_REL):
        w = coeff_ref[r, 0] * bases_ref[0]
        for b in range(1, NUM_BASES):
            w = w + coeff_ref[r, b] * bases_ref[b]
        z_ref[r] = jnp.dot(xb, w, preferred_element_type=jnp.float32)


def _compute_z(coefficients, bases, x):
    return pl.pallas_call(
        _z_body,
        grid=(N_NODES // NODE_BLK,),
        in_specs=[
            pl.BlockSpec((NUM_REL, NUM_BASES), lambda i: (0, 0)),
            pl.BlockSpec((NUM_BASES, F, F), lambda i: (0, 0, 0)),
            pl.BlockSpec((NODE_BLK, F), lambda i: (i, 0)),
        ],
        out_specs=pl.BlockSpec((NUM_REL, NODE_BLK, F), lambda i: (0, i, 0)),
        out_shape=jax.ShapeDtypeStruct((NUM_REL, N_NODES, F), jnp.float32),
    )(coefficients, bases, x)


def _split_body(ei_ref, row_ref, col_ref):
    row_ref[...] = ei_ref[0]
    col_ref[...] = ei_ref[1]


def _split_edges(ei):
    return pl.pallas_call(
        _split_body,
        out_shape=[
            jax.ShapeDtypeStruct((N_EDGES,), jnp.int32),
            jax.ShapeDtypeStruct((N_EDGES,), jnp.int32),
        ],
    )(ei)


def _sc_body(z_hbm, row_hbm, col_hbm, et_hbm, zeros_hbm, out_hbm,
             row_v, src_v, et_v, dst0, dst1, rows0, rows1, acc, sem0, sem1):
    cid = lax.axis_index("c")
    sid = lax.axis_index("s")
    wid = sid * NC + cid

    pltpu.sync_copy(zeros_hbm.at[pl.ds(sid * ROWS_MAIN, ROWS_MAIN)],
                    acc.at[pl.ds(sid * ROWS_MAIN, ROWS_MAIN)])

    @pl.when(sid == 0)
    def _zero_tail():
        pltpu.sync_copy(zeros_hbm.at[pl.ds(NS * ROWS_MAIN, ROWS_TAIL)],
                        acc.at[pl.ds(NS * ROWS_MAIN, ROWS_TAIL)])

    plsc.subcore_barrier()

    base = wid * EDGES_PER_W
    pltpu.sync_copy(row_hbm.at[pl.ds(base, EDGES_PER_W)], row_v)
    pltpu.sync_copy(col_hbm.at[pl.ds(base, EDGES_PER_W)], src_v)

    def et_block(bk, carry):
        pltpu.sync_copy(et_hbm.at[pl.ds(base + bk * ET_BLK, ET_BLK)], et_v)

        def idx_body(j, c):
            o = bk * ET_BLK + j * 16
            src_v[pl.ds(o, 16)] = (et_v[pl.ds(j * 16, 16)] * N_NODES
                                   + src_v[pl.ds(o, 16)])
            return c

        lax.fori_loop(0, ET_BLK // 16, idx_body, 0)
        return carry

    lax.fori_loop(0, EDGES_PER_W // ET_BLK, et_block, 0)

    def start_gather(k, buf, sem):
        pltpu.async_copy(z_hbm.at[src_v.at[pl.ds(k * CHUNK, CHUNK)]], buf, sem)

    def wait_gather(buf, sem):
        pltpu.make_async_copy(z_hbm.at[src_v.at[pl.ds(0, CHUNK)]],
                              buf, sem).wait()

    def build_dst(k, dst):
        for i in range(CHUNK // 16):
            dst[pl.ds(i * 16, 16)] = row_v[pl.ds(k * CHUNK + i * 16, 16)]

    def scatter(buf, dst):
        pltpu.sync_copy(buf, acc.at[dst], add=True)

    start_gather(0, rows0, sem0)

    def pair(k0):
        start_gather(k0 + 1, rows1, sem1)
        build_dst(k0, dst0)
        wait_gather(rows0, sem0)
        scatter(rows0, dst0)
        start_gather(k0 + 2, rows0, sem0)
        build_dst(k0 + 1, dst1)
        wait_gather(rows1, sem1)
        scatter(rows1, dst1)

    def pipe_body(g, carry):
        k0 = 4 * g
        pair(k0)
        pair(k0 + 2)
        return carry

    lax.fori_loop(0, (N_CHUNKS - 1) // 4, pipe_body, 0)
    build_dst(N_CHUNKS - 1, dst0)
    wait_gather(rows0, sem0)
    scatter(rows0, dst0)

    plsc.subcore_barrier()
    pltpu.sync_copy(acc.at[pl.ds(sid * ROWS_MAIN, ROWS_MAIN)],
                    out_hbm.at[cid].at[pl.ds(sid * ROWS_MAIN, ROWS_MAIN)])

    @pl.when(sid == 0)
    def _dump_tail():
        pltpu.sync_copy(acc.at[pl.ds(NS * ROWS_MAIN, ROWS_TAIL)],
                        out_hbm.at[cid].at[pl.ds(NS * ROWS_MAIN, ROWS_TAIL)])


@functools.cache
def _sc_scatter():
    return pl.kernel(
        _sc_body,
        out_type=jax.ShapeDtypeStruct((NC, N_NODES, F), jnp.float32),
        mesh=plsc.VectorSubcoreMesh(core_axis_name="c", subcore_axis_name="s"),
        scratch_types=[
            pltpu.VMEM((EDGES_PER_W,), jnp.int32),
            pltpu.VMEM((EDGES_PER_W,), jnp.int32),
            pltpu.VMEM((ET_BLK,), jnp.int32),
            pltpu.VMEM((CHUNK,), jnp.int32),
            pltpu.VMEM((CHUNK,), jnp.int32),
            pltpu.VMEM((CHUNK, F), jnp.float32),
            pltpu.VMEM((CHUNK, F), jnp.float32),
            pltpu.VMEM_SHARED((N_NODES, F), jnp.float32),
            pltpu.SemaphoreType.DMA,
            pltpu.SemaphoreType.DMA,
        ],
    )


def _combine_body(x_ref, ws_ref, p_ref, o_ref):
    o_ref[...] = (lax.dot_general(x_ref[...], ws_ref[...],
                                  (((1,), (1,)), ((), ())),
                                  preferred_element_type=jnp.float32)
                  + p_ref[0] + p_ref[1])


def _combine(x, w_self, partials):
    return pl.pallas_call(
        _combine_body,
        grid=(N_NODES // NODE_BLK,),
        in_specs=[
            pl.BlockSpec((NODE_BLK, F), lambda i: (i, 0)),
            pl.BlockSpec((F, F), lambda i: (0, 0)),
            pl.BlockSpec((NC, NODE_BLK, F), lambda i: (0, i, 0)),
        ],
        out_specs=pl.BlockSpec((NODE_BLK, F), lambda i: (i, 0)),
        out_shape=jax.ShapeDtypeStruct((N_NODES, F), jnp.float32),
    )(x, w_self, partials)


def kernel(x, edge_index, edge_type, bases, coefficients, W_self):
    ei = edge_index.astype(jnp.int32)
    et = edge_type.astype(jnp.int32)
    row, col = _split_edges(ei)
    z = _compute_z(coefficients, bases, x).reshape(NUM_REL * N_NODES, F)
    zeros = jnp.zeros((N_NODES, F), jnp.float32)
    partials = _sc_scatter()(z, row, col, et, zeros)
    return _combine(x, W_self, partials)


# speedup vs baseline: 1.0005x; 1.0005x over previous
import functools

import jax
import jax.numpy as jnp
from jax import lax
from jax.experimental import pallas as pl
from jax.experimental.pallas import tpu as pltpu
from jax.experimental.pallas import tpu_sc as plsc

N_NODES = 10000
N_EDGES = 320000
F = 128
NUM_REL = 8
NUM_BASES = 4

NODE_BLK = 1000

NC = 2
NS = 16
NW = NC * NS
EDGES_PER_W = N_EDGES // NW
CHUNK = 80
N_CHUNKS = EDGES_PER_W // CHUNK
ROWS_MAIN = 624
ROWS_TAIL = N_NODES - NS * ROWS_MAIN
ET_BLK = 2000


def _z_body(coeff_ref, bases_ref, x_ref, z_ref):
    xb = x_ref[...]
    for r in range(NUM_REL):
        w = coeff_ref[r, 0] * bases_ref[0]
        for b in range(1, NUM_BASES):
            w = w + coeff_ref[r, b] * bases_ref[b]
        z_ref[r] = jnp.dot(xb, w, preferred_element_type=jnp.float32)


def _compute_z(coefficients, bases, x):
    return pl.pallas_call(
        _z_body,
        grid=(N_NODES // NODE_BLK,),
        in_specs=[
            pl.BlockSpec((NUM_REL, NUM_BASES), lambda i: (0, 0)),
            pl.BlockSpec((NUM_BASES, F, F), lambda i: (0, 0, 0)),
            pl.BlockSpec((NODE_BLK, F), lambda i: (i, 0)),
        ],
        out_specs=pl.BlockSpec((NUM_REL, NODE_BLK, F), lambda i: (0, i, 0)),
        out_shape=jax.ShapeDtypeStruct((NUM_REL, N_NODES, F), jnp.float32),
    )(coefficients, bases, x)


def _split_body(ei_ref, row_ref, col_ref):
    row_ref[...] = ei_ref[0]
    col_ref[...] = ei_ref[1]


def _split_edges(ei):
    return pl.pallas_call(
        _split_body,
        out_shape=[
            jax.ShapeDtypeStruct((N_EDGES,), jnp.int32),
            jax.ShapeDtypeStruct((N_EDGES,), jnp.int32),
        ],
    )(ei)


def _sc_body(z_hbm, row_hbm, col_hbm, et_hbm, zeros_hbm, out_hbm,
             row_v, src_v, et_v, dst0, dst1, rows0, rows1, acc, sem0, sem1):
    cid = lax.axis_index("c")
    sid = lax.axis_index("s")
    wid = sid * NC + cid

    scope = jax.named_scope
    with scope("acc_zero"):
        pltpu.sync_copy(zeros_hbm.at[pl.ds(sid * ROWS_MAIN, ROWS_MAIN)],
                        acc.at[pl.ds(sid * ROWS_MAIN, ROWS_MAIN)])

    @pl.when(sid == 0)
    def _zero_tail():
        pltpu.sync_copy(zeros_hbm.at[pl.ds(NS * ROWS_MAIN, ROWS_TAIL)],
                        acc.at[pl.ds(NS * ROWS_MAIN, ROWS_TAIL)])

    plsc.subcore_barrier()

    base = wid * EDGES_PER_W
    with scope("stage_edges"):
        pltpu.sync_copy(row_hbm.at[pl.ds(base, EDGES_PER_W)], row_v)
        pltpu.sync_copy(col_hbm.at[pl.ds(base, EDGES_PER_W)], src_v)

    def et_block(bk, carry):
        pltpu.sync_copy(et_hbm.at[pl.ds(base + bk * ET_BLK, ET_BLK)], et_v)

        def idx_body(j, c):
            o = bk * ET_BLK + j * 16
            src_v[pl.ds(o, 16)] = (et_v[pl.ds(j * 16, 16)] * N_NODES
                                   + src_v[pl.ds(o, 16)])
            return c

        lax.fori_loop(0, ET_BLK // 16, idx_body, 0)
        return carry

    with scope("idx_prologue"):
        lax.fori_loop(0, EDGES_PER_W // ET_BLK, et_block, 0)

    def start_gather(k, buf, sem):
        pltpu.async_copy(z_hbm.at[src_v.at[pl.ds(k * CHUNK, CHUNK)]], buf, sem)

    def wait_gather(buf, sem):
        pltpu.make_async_copy(z_hbm.at[src_v.at[pl.ds(0, CHUNK)]],
                              buf, sem).wait()

    def build_dst(k, dst):
        for i in range(CHUNK // 16):
            dst[pl.ds(i * 16, 16)] = row_v[pl.ds(k * CHUNK + i * 16, 16)]

    def scatter(buf, dst):
        pltpu.sync_copy(buf, acc.at[dst], add=True)

    start_gather(0, rows0, sem0)

    def pair(k0):
        start_gather(k0 + 1, rows1, sem1)
        build_dst(k0, dst0)
        wait_gather(rows0, sem0)
        scatter(rows0, dst0)
        start_gather(k0 + 2, rows0, sem0)
        build_dst(k0 + 1, dst1)
        wait_gather(rows1, sem1)
        scatter(rows1, dst1)

    def pipe_body(g, carry):
        k0 = 4 * g
        pair(k0)
        pair(k0 + 2)
        return carry

    with scope("edge_loop"):
        lax.fori_loop(0, (N_CHUNKS - 1) // 4, pipe_body, 0)
        build_dst(N_CHUNKS - 1, dst0)
        wait_gather(rows0, sem0)
        scatter(rows0, dst0)

    plsc.subcore_barrier()
    with scope("dump"):
        pltpu.sync_copy(acc.at[pl.ds(sid * ROWS_MAIN, ROWS_MAIN)],
                        out_hbm.at[cid].at[pl.ds(sid * ROWS_MAIN, ROWS_MAIN)])

    @pl.when(sid == 0)
    def _dump_tail():
        pltpu.sync_copy(acc.at[pl.ds(NS * ROWS_MAIN, ROWS_TAIL)],
                        out_hbm.at[cid].at[pl.ds(NS * ROWS_MAIN, ROWS_TAIL)])


@functools.cache
def _sc_scatter():
    return pl.kernel(
        _sc_body,
        out_type=jax.ShapeDtypeStruct((NC, N_NODES, F), jnp.float32),
        mesh=plsc.VectorSubcoreMesh(core_axis_name="c", subcore_axis_name="s"),
        scratch_types=[
            pltpu.VMEM((EDGES_PER_W,), jnp.int32),
            pltpu.VMEM((EDGES_PER_W,), jnp.int32),
            pltpu.VMEM((ET_BLK,), jnp.int32),
            pltpu.VMEM((CHUNK,), jnp.int32),
            pltpu.VMEM((CHUNK,), jnp.int32),
            pltpu.VMEM((CHUNK, F), jnp.float32),
            pltpu.VMEM((CHUNK, F), jnp.float32),
            pltpu.VMEM_SHARED((N_NODES, F), jnp.float32),
            pltpu.SemaphoreType.DMA,
            pltpu.SemaphoreType.DMA,
        ],
    )


def _combine_body(x_ref, ws_ref, p_ref, o_ref):
    o_ref[...] = (lax.dot_general(x_ref[...], ws_ref[...],
                                  (((1,), (1,)), ((), ())),
                                  preferred_element_type=jnp.float32)
                  + p_ref[0] + p_ref[1])


def _combine(x, w_self, partials):
    return pl.pallas_call(
        _combine_body,
        grid=(N_NODES // NODE_BLK,),
        in_specs=[
            pl.BlockSpec((NODE_BLK, F), lambda i: (i, 0)),
            pl.BlockSpec((F, F), lambda i: (0, 0)),
            pl.BlockSpec((NC, NODE_BLK, F), lambda i: (0, i, 0)),
        ],
        out_specs=pl.BlockSpec((NODE_BLK, F), lambda i: (i, 0)),
        out_shape=jax.ShapeDtypeStruct((N_NODES, F), jnp.float32),
    )(x, w_self, partials)


def kernel(x, edge_index, edge_type, bases, coefficients, W_self):
    ei = edge_index.astype(jnp.int32)
    et = edge_type.astype(jnp.int32)
    row, col = _split_edges(ei)
    z = _compute_z(coefficients, bases, x).reshape(NUM_REL * N_NODES, F)
    zeros = jnp.zeros((N_NODES, F), jnp.float32)
    partials = _sc_scatter()(z, row, col, et, zeros)
    return _combine(x, W_self, partials)


# speedup vs baseline: 1.0310x; 1.0305x over previous
import functools

import jax
import jax.numpy as jnp
from jax import lax
from jax.experimental import pallas as pl
from jax.experimental.pallas import tpu as pltpu
from jax.experimental.pallas import tpu_sc as plsc

N_NODES = 10000
N_EDGES = 320000
F = 128
NUM_REL = 8
NUM_BASES = 4

NODE_BLK = 1000

NC = 2
NS = 16
NW = NC * NS
EDGES_PER_W = N_EDGES // NW
CHUNK = 80
N_CHUNKS = EDGES_PER_W // CHUNK
ROWS_MAIN = 624
ROWS_TAIL = N_NODES - NS * ROWS_MAIN
ET_BLK = 2000


def _z_body(coeff_ref, bases_ref, x_ref, z_ref):
    xb = x_ref[...]
    for r in range(NUM_REL):
        w = coeff_ref[r, 0] * bases_ref[0]
        for b in range(1, NUM_BASES):
            w = w + coeff_ref[r, b] * bases_ref[b]
        z_ref[r] = jnp.dot(xb, w, preferred_element_type=jnp.float32)


def _compute_z(coefficients, bases, x):
    return pl.pallas_call(
        _z_body,
        grid=(N_NODES // NODE_BLK,),
        in_specs=[
            pl.BlockSpec((NUM_REL, NUM_BASES), lambda i: (0, 0)),
            pl.BlockSpec((NUM_BASES, F, F), lambda i: (0, 0, 0)),
            pl.BlockSpec((NODE_BLK, F), lambda i: (i, 0)),
        ],
        out_specs=pl.BlockSpec((NUM_REL, NODE_BLK, F), lambda i: (0, i, 0)),
        out_shape=jax.ShapeDtypeStruct((NUM_REL, N_NODES, F), jnp.float32),
    )(coefficients, bases, x)


def _split_body(ei_ref, row_ref, col_ref):
    row_ref[...] = ei_ref[0]
    col_ref[...] = ei_ref[1]


def _split_edges(ei):
    return pl.pallas_call(
        _split_body,
        out_shape=[
            jax.ShapeDtypeStruct((N_EDGES,), jnp.int32),
            jax.ShapeDtypeStruct((N_EDGES,), jnp.int32),
        ],
    )(ei)


def _sc_body(z_hbm, row_hbm, col_hbm, et_hbm, zeros_hbm, out_hbm,
             row_v, src_v, et_v, dst0, dst1, rows0, rows1, acc,
             sem0, sem1, zsem):
    cid = lax.axis_index("c")
    sid = lax.axis_index("s")
    wid = sid * NC + cid

    scope = jax.named_scope
    pltpu.async_copy(zeros_hbm.at[pl.ds(sid * ROWS_MAIN, ROWS_MAIN)],
                     acc.at[pl.ds(sid * ROWS_MAIN, ROWS_MAIN)], zsem)

    @pl.when(sid == 0)
    def _zero_tail():
        pltpu.async_copy(zeros_hbm.at[pl.ds(NS * ROWS_MAIN, ROWS_TAIL)],
                         acc.at[pl.ds(NS * ROWS_MAIN, ROWS_TAIL)], zsem)

    base = wid * EDGES_PER_W
    with scope("stage_edges"):
        pltpu.sync_copy(row_hbm.at[pl.ds(base, EDGES_PER_W)], row_v)
        pltpu.sync_copy(col_hbm.at[pl.ds(base, EDGES_PER_W)], src_v)

    def et_block(bk, carry):
        pltpu.sync_copy(et_hbm.at[pl.ds(base + bk * ET_BLK, ET_BLK)], et_v)

        def idx_body(j, c):
            for i in range(CHUNK // 16):
                o = j * CHUNK + i * 16
                src_v[pl.ds(bk * ET_BLK + o, 16)] = (
                    et_v[pl.ds(o, 16)] * N_NODES
                    + src_v[pl.ds(bk * ET_BLK + o, 16)])
            return c

        lax.fori_loop(0, ET_BLK // CHUNK, idx_body, 0)
        return carry

    with scope("idx_prologue"):
        lax.fori_loop(0, EDGES_PER_W // ET_BLK, et_block, 0)

    def start_gather(k, buf, sem):
        pltpu.async_copy(z_hbm.at[src_v.at[pl.ds(k * CHUNK, CHUNK)]], buf, sem)

    def wait_gather(buf, sem):
        pltpu.make_async_copy(z_hbm.at[src_v.at[pl.ds(0, CHUNK)]],
                              buf, sem).wait()

    def build_dst(k, dst):
        for i in range(CHUNK // 16):
            dst[pl.ds(i * 16, 16)] = row_v[pl.ds(k * CHUNK + i * 16, 16)]

    def scatter(buf, dst):
        pltpu.sync_copy(buf, acc.at[dst], add=True)

    start_gather(0, rows0, sem0)
    pltpu.make_async_copy(zeros_hbm.at[pl.ds(sid * ROWS_MAIN, ROWS_MAIN)],
                          acc.at[pl.ds(sid * ROWS_MAIN, ROWS_MAIN)],
                          zsem).wait()

    @pl.when(sid == 0)
    def _zero_tail_wait():
        pltpu.make_async_copy(zeros_hbm.at[pl.ds(NS * ROWS_MAIN, ROWS_TAIL)],
                              acc.at[pl.ds(NS * ROWS_MAIN, ROWS_TAIL)],
                              zsem).wait()

    plsc.subcore_barrier()


    def pair(k0):
        start_gather(k0 + 1, rows1, sem1)
        build_dst(k0, dst0)
        wait_gather(rows0, sem0)
        scatter(rows0, dst0)
        start_gather(k0 + 2, rows0, sem0)
        build_dst(k0 + 1, dst1)
        wait_gather(rows1, sem1)
        scatter(rows1, dst1)

    def pipe_body(g, carry):
        k0 = 4 * g
        pair(k0)
        pair(k0 + 2)
        return carry

    with scope("edge_loop"):
        lax.fori_loop(0, (N_CHUNKS - 1) // 4, pipe_body, 0)
        build_dst(N_CHUNKS - 1, dst0)
        wait_gather(rows0, sem0)
        scatter(rows0, dst0)

    plsc.subcore_barrier()
    with scope("dump"):
        pltpu.sync_copy(acc.at[pl.ds(sid * ROWS_MAIN, ROWS_MAIN)],
                        out_hbm.at[cid].at[pl.ds(sid * ROWS_MAIN, ROWS_MAIN)])

    @pl.when(sid == 0)
    def _dump_tail():
        pltpu.sync_copy(acc.at[pl.ds(NS * ROWS_MAIN, ROWS_TAIL)],
                        out_hbm.at[cid].at[pl.ds(NS * ROWS_MAIN, ROWS_TAIL)])


@functools.cache
def _sc_scatter():
    return pl.kernel(
        _sc_body,
        out_type=jax.ShapeDtypeStruct((NC, N_NODES, F), jnp.float32),
        mesh=plsc.VectorSubcoreMesh(core_axis_name="c", subcore_axis_name="s"),
        scratch_types=[
            pltpu.VMEM((EDGES_PER_W,), jnp.int32),
            pltpu.VMEM((EDGES_PER_W,), jnp.int32),
            pltpu.VMEM((ET_BLK,), jnp.int32),
            pltpu.VMEM((CHUNK,), jnp.int32),
            pltpu.VMEM((CHUNK,), jnp.int32),
            pltpu.VMEM((CHUNK, F), jnp.float32),
            pltpu.VMEM((CHUNK, F), jnp.float32),
            pltpu.VMEM_SHARED((N_NODES, F), jnp.float32),
            pltpu.SemaphoreType.DMA,
            pltpu.SemaphoreType.DMA,
            pltpu.SemaphoreType.DMA,
        ],
    )


def _combine_body(x_ref, ws_ref, p_ref, o_ref):
    o_ref[...] = (lax.dot_general(x_ref[...], ws_ref[...],
                                  (((1,), (1,)), ((), ())),
                                  preferred_element_type=jnp.float32)
                  + p_ref[0] + p_ref[1])


def _combine(x, w_self, partials):
    return pl.pallas_call(
        _combine_body,
        grid=(N_NODES // NODE_BLK,),
        in_specs=[
            pl.BlockSpec((NODE_BLK, F), lambda i: (i, 0)),
            pl.BlockSpec((F, F), lambda i: (0, 0)),
            pl.BlockSpec((NC, NODE_BLK, F), lambda i: (0, i, 0)),
        ],
        out_specs=pl.BlockSpec((NODE_BLK, F), lambda i: (i, 0)),
        out_shape=jax.ShapeDtypeStruct((N_NODES, F), jnp.float32),
    )(x, w_self, partials)


def kernel(x, edge_index, edge_type, bases, coefficients, W_self):
    ei = edge_index.astype(jnp.int32)
    et = edge_type.astype(jnp.int32)
    row, col = _split_edges(ei)
    z = _compute_z(coefficients, bases, x).reshape(NUM_REL * N_NODES, F)
    zeros = jnp.zeros((N_NODES, F), jnp.float32)
    partials = _sc_scatter()(z, row, col, et, zeros)
    return _combine(x, W_self, partials)


# speedup vs baseline: 1.0727x; 1.0405x over previous
import functools

import jax
import jax.numpy as jnp
from jax import lax
from jax.experimental import pallas as pl
from jax.experimental.pallas import tpu as pltpu
from jax.experimental.pallas import tpu_sc as plsc

N_NODES = 10000
N_EDGES = 320000
F = 128
NUM_REL = 8
NUM_BASES = 4

NODE_BLK = 2000

NC = 2
NS = 16
NW = NC * NS
EDGES_PER_W = N_EDGES // NW
CHUNK = 80
N_CHUNKS = EDGES_PER_W // CHUNK
ROWS_MAIN = 624
ROWS_TAIL = N_NODES - NS * ROWS_MAIN
ET_BLK = 2000


def _z_body(coeff_ref, bases_ref, x_ref, z_ref):
    xb = x_ref[...]
    for r in range(NUM_REL):
        w = coeff_ref[r, 0] * bases_ref[0]
        for b in range(1, NUM_BASES):
            w = w + coeff_ref[r, b] * bases_ref[b]
        z_ref[r] = jnp.dot(xb, w, preferred_element_type=jnp.float32)


def _compute_z(coefficients, bases, x):
    return pl.pallas_call(
        _z_body,
        grid=(N_NODES // NODE_BLK,),
        in_specs=[
            pl.BlockSpec((NUM_REL, NUM_BASES), lambda i: (0, 0)),
            pl.BlockSpec((NUM_BASES, F, F), lambda i: (0, 0, 0)),
            pl.BlockSpec((NODE_BLK, F), lambda i: (i, 0)),
        ],
        out_specs=pl.BlockSpec((NUM_REL, NODE_BLK, F), lambda i: (0, i, 0)),
        out_shape=jax.ShapeDtypeStruct((NUM_REL, N_NODES, F), jnp.float32),
    )(coefficients, bases, x)


def _split_body(ei_ref, row_ref, col_ref):
    row_ref[...] = ei_ref[0]
    col_ref[...] = ei_ref[1]


def _split_edges(ei):
    return pl.pallas_call(
        _split_body,
        out_shape=[
            jax.ShapeDtypeStruct((N_EDGES,), jnp.int32),
            jax.ShapeDtypeStruct((N_EDGES,), jnp.int32),
        ],
    )(ei)


def _sc_body(z_hbm, row_hbm, col_hbm, et_hbm, zeros_hbm, out_hbm,
             row_v, src_v, et_v, dst0, dst1, rows0, rows1, acc,
             sem0, sem1, zsem):
    cid = lax.axis_index("c")
    sid = lax.axis_index("s")
    wid = sid * NC + cid

    scope = jax.named_scope
    pltpu.async_copy(zeros_hbm.at[pl.ds(sid * ROWS_MAIN, ROWS_MAIN)],
                     acc.at[pl.ds(sid * ROWS_MAIN, ROWS_MAIN)], zsem)

    @pl.when(sid == 0)
    def _zero_tail():
        pltpu.async_copy(zeros_hbm.at[pl.ds(NS * ROWS_MAIN, ROWS_TAIL)],
                         acc.at[pl.ds(NS * ROWS_MAIN, ROWS_TAIL)], zsem)

    base = wid * EDGES_PER_W
    with scope("stage_edges"):
        pltpu.sync_copy(row_hbm.at[pl.ds(base, EDGES_PER_W)], row_v)
        pltpu.sync_copy(col_hbm.at[pl.ds(base, EDGES_PER_W)], src_v)

    def et_block(bk, carry):
        pltpu.sync_copy(et_hbm.at[pl.ds(base + bk * ET_BLK, ET_BLK)], et_v)

        def idx_body(j, c):
            for i in range(CHUNK // 16):
                o = j * CHUNK + i * 16
                src_v[pl.ds(bk * ET_BLK + o, 16)] = (
                    et_v[pl.ds(o, 16)] * N_NODES
                    + src_v[pl.ds(bk * ET_BLK + o, 16)])
            return c

        lax.fori_loop(0, ET_BLK // CHUNK, idx_body, 0)
        return carry

    with scope("idx_prologue"):
        lax.fori_loop(0, EDGES_PER_W // ET_BLK, et_block, 0)

    def start_gather(k, buf, sem):
        pltpu.async_copy(z_hbm.at[src_v.at[pl.ds(k * CHUNK, CHUNK)]], buf, sem)

    def wait_gather(buf, sem):
        pltpu.make_async_copy(z_hbm.at[src_v.at[pl.ds(0, CHUNK)]],
                              buf, sem).wait()

    def build_dst(k, dst):
        for i in range(CHUNK // 16):
            dst[pl.ds(i * 16, 16)] = row_v[pl.ds(k * CHUNK + i * 16, 16)]

    def scatter(buf, dst):
        pltpu.sync_copy(buf, acc.at[dst], add=True)

    start_gather(0, rows0, sem0)
    pltpu.make_async_copy(zeros_hbm.at[pl.ds(sid * ROWS_MAIN, ROWS_MAIN)],
                          acc.at[pl.ds(sid * ROWS_MAIN, ROWS_MAIN)],
                          zsem).wait()

    @pl.when(sid == 0)
    def _zero_tail_wait():
        pltpu.make_async_copy(zeros_hbm.at[pl.ds(NS * ROWS_MAIN, ROWS_TAIL)],
                              acc.at[pl.ds(NS * ROWS_MAIN, ROWS_TAIL)],
                              zsem).wait()

    plsc.subcore_barrier()


    def pair(k0):
        start_gather(k0 + 1, rows1, sem1)
        build_dst(k0, dst0)
        wait_gather(rows0, sem0)
        scatter(rows0, dst0)
        start_gather(k0 + 2, rows0, sem0)
        build_dst(k0 + 1, dst1)
        wait_gather(rows1, sem1)
        scatter(rows1, dst1)

    def pipe_body(g, carry):
        k0 = 4 * g
        pair(k0)
        pair(k0 + 2)
        return carry

    with scope("edge_loop"):
        lax.fori_loop(0, (N_CHUNKS - 1) // 4, pipe_body, 0)
        build_dst(N_CHUNKS - 1, dst0)
        wait_gather(rows0, sem0)
        scatter(rows0, dst0)

    plsc.subcore_barrier()
    with scope("dump"):
        pltpu.sync_copy(acc.at[pl.ds(sid * ROWS_MAIN, ROWS_MAIN)],
                        out_hbm.at[cid].at[pl.ds(sid * ROWS_MAIN, ROWS_MAIN)])

    @pl.when(sid == 0)
    def _dump_tail():
        pltpu.sync_copy(acc.at[pl.ds(NS * ROWS_MAIN, ROWS_TAIL)],
                        out_hbm.at[cid].at[pl.ds(NS * ROWS_MAIN, ROWS_TAIL)])


@functools.cache
def _sc_scatter():
    return pl.kernel(
        _sc_body,
        out_type=jax.ShapeDtypeStruct((NC, N_NODES, F), jnp.float32),
        mesh=plsc.VectorSubcoreMesh(core_axis_name="c", subcore_axis_name="s"),
        scratch_types=[
            pltpu.VMEM((EDGES_PER_W,), jnp.int32),
            pltpu.VMEM((EDGES_PER_W,), jnp.int32),
            pltpu.VMEM((ET_BLK,), jnp.int32),
            pltpu.VMEM((CHUNK,), jnp.int32),
            pltpu.VMEM((CHUNK,), jnp.int32),
            pltpu.VMEM((CHUNK, F), jnp.float32),
            pltpu.VMEM((CHUNK, F), jnp.float32),
            pltpu.VMEM_SHARED((N_NODES, F), jnp.float32),
            pltpu.SemaphoreType.DMA,
            pltpu.SemaphoreType.DMA,
            pltpu.SemaphoreType.DMA,
        ],
    )


def _combine_body(x_ref, ws_ref, p_ref, o_ref):
    o_ref[...] = (lax.dot_general(x_ref[...], ws_ref[...],
                                  (((1,), (1,)), ((), ())),
                                  preferred_element_type=jnp.float32)
                  + p_ref[0] + p_ref[1])


def _combine(x, w_self, partials):
    return pl.pallas_call(
        _combine_body,
        grid=(N_NODES // NODE_BLK,),
        in_specs=[
            pl.BlockSpec((NODE_BLK, F), lambda i: (i, 0)),
            pl.BlockSpec((F, F), lambda i: (0, 0)),
            pl.BlockSpec((NC, NODE_BLK, F), lambda i: (0, i, 0)),
        ],
        out_specs=pl.BlockSpec((NODE_BLK, F), lambda i: (i, 0)),
        out_shape=jax.ShapeDtypeStruct((N_NODES, F), jnp.float32),
    )(x, w_self, partials)


def kernel(x, edge_index, edge_type, bases, coefficients, W_self):
    ei = edge_index.astype(jnp.int32)
    et = edge_type.astype(jnp.int32)
    row, col = _split_edges(ei)
    z = _compute_z(coefficients, bases, x).reshape(NUM_REL * N_NODES, F)
    zeros = jnp.zeros((N_NODES, F), jnp.float32)
    partials = _sc_scatter()(z, row, col, et, zeros)
    return _combine(x, W_self, partials)
